# baseline (speedup 1.0000x reference)
import jax
import jax.numpy as jnp
import numpy as np
from jax import lax
from jax.experimental import pallas as pl
from jax.experimental.pallas import tpu as pltpu
from jax.experimental.pallas import tpu_sc as plsc

NC = 2
NS = 16
L = 16

B = 4096
N = 20
D = 32
NPROD = N + N * (N - 1) // 2
OUT_F = NPROD * D
NW = NC * NS
BPW = B // NW
NSLOT = N + 1
EROWS = NSLOT * D
TCOL = 512
PCHUNK = 30
NPC = NPROD // PCHUNK


def _pair_tables():
    pa, pb = [], []
    for i in range(N):
        pa.append(N)
        pb.append(i)
    for i in range(N):
        for j in range(i + 1, N):
            pa.append(i)
            pb.append(j)
    return pa, pb


def _t_body(src_ref, eye_ref, dst_ref):
    dst_ref[...] = lax.dot_general(
        src_ref[...], eye_ref[...], (((0,), (0,)), ((), ())),
        preferred_element_type=jnp.float32)


def _transpose_table(tabT, eye):
    m = tabT.shape[1]
    grid = (m + TCOL - 1) // TCOL
    return pl.pallas_call(
        _t_body,
        grid=(grid,),
        in_specs=[pl.BlockSpec((D, TCOL), lambda c: (0, c)),
                  pl.BlockSpec((D, D), lambda c: (0, 0))],
        out_specs=pl.BlockSpec((TCOL, D), lambda c: (c, 0)),
        out_shape=jax.ShapeDtypeStruct((m, D), jnp.float32),
    )(tabT, eye)


def _g_body(uidx_hbm, midxT_hbm, utab_hbm, itab_hbm, emb_hbm,
            midx_v, uidx_v, rows_v, ev_v, sem_g):
    w = lax.axis_index("s") * NC + lax.axis_index("c")
    lane0 = w * BPW

    pltpu.sync_copy(midxT_hbm.at[:, pl.ds(lane0, BPW)], midx_v)
    pltpu.sync_copy(uidx_hbm.at[pl.ds(lane0, BPW)], uidx_v)

    lane_iota = jnp.arange(L, dtype=jnp.int32)
    gsize = NSLOT // 3

    for g in range(3):
        for sl in range(gsize):
            s = g * gsize + sl
            if s < N:
                pltpu.async_copy(itab_hbm.at[midx_v.at[s]],
                                 rows_v.at[pl.ds(sl * BPW, BPW)], sem_g)
            else:
                pltpu.async_copy(utab_hbm.at[uidx_v],
                                 rows_v.at[pl.ds(sl * BPW, BPW)], sem_g)
        for _ in range(gsize):
            pltpu.make_async_copy(
                itab_hbm.at[midx_v.at[0]],
                rows_v.at[pl.ds(0, BPW)], sem_g).wait()

        def dbody(d, _, g=g):
            for sl in range(gsize):
                for blk in range(BPW // L):
                    rows = lane_iota + (sl * BPW + blk * L)
                    vals = plsc.load_gather(
                        rows_v, [rows, jnp.full((L,), 0, jnp.int32) + d])
                    ev_v[(g * gsize + sl) * D + d, pl.ds(blk * L, L)] = vals
            return _

        lax.fori_loop(0, D, dbody, None)

    pltpu.sync_copy(ev_v, emb_hbm.at[:, pl.ds(lane0, BPW)])


def _gather_emb(uidx, midxT, utab_rm, itab_rm):
    mesh = plsc.VectorSubcoreMesh(core_axis_name="c", subcore_axis_name="s")
    k = pl.kernel(
        _g_body,
        out_type=jax.ShapeDtypeStruct((EROWS, B), jnp.float32),
        mesh=mesh,
        compiler_params=pltpu.CompilerParams(use_tc_tiling_on_sc=False,
                                             needs_layout_passes=False),
        scratch_types=[
            pltpu.VMEM((N, BPW), jnp.int32),
            pltpu.VMEM((BPW,), jnp.int32),
            pltpu.VMEM((NSLOT // 3 * BPW, D), jnp.float32),
            pltpu.VMEM((EROWS, BPW), jnp.float32),
            pltpu.SemaphoreType.DMA,
        ],
    )
    return k(uidx, midxT, utab_rm, itab_rm)


def _p_body(pa_ref, pb_ref, emb_ref, wrow_ref, out_ref):
    pc = pl.program_id(1)
    for k in range(PCHUNK):
        p = pc * PCHUNK + k
        a0 = pa_ref[p] * D
        b0 = pb_ref[p] * D
        av = emb_ref[pl.ds(a0, D), :]
        bv = emb_ref[pl.ds(b0, D), :]
        out_ref[pl.ds(k * D, D), :] = av * bv * wrow_ref[pl.ds(k * D, D), :]


def _products(emb, wrow, pa_arr, pb_arr):
    grid_spec = pltpu.PrefetchScalarGridSpec(
        num_scalar_prefetch=2,
        grid=(B // TCOL, NPC),
        in_specs=[
            pl.BlockSpec((EROWS, TCOL), lambda bb, pc, *_: (0, bb)),
            pl.BlockSpec((PCHUNK * D, 1), lambda bb, pc, *_: (pc, 0)),
        ],
        out_specs=pl.BlockSpec((PCHUNK * D, TCOL),
                               lambda bb, pc, *_: (pc, bb)),
    )
    return pl.pallas_call(
        _p_body,
        grid_spec=grid_spec,
        out_shape=jax.ShapeDtypeStruct((OUT_F, B), jnp.float32),
    )(pa_arr, pb_arr, emb, wrow)


def kernel(user, memory, user_table, item_table, weights):
    uidx = user.reshape(-1).astype(jnp.int32)
    midxT = memory.T.astype(jnp.int32)

    eye = jnp.eye(D, dtype=jnp.float32)
    utab_rm = _transpose_table(user_table.astype(jnp.float32).T, eye)
    itab_rm = _transpose_table(item_table.astype(jnp.float32).T, eye)

    emb = _gather_emb(uidx, midxT, utab_rm, itab_rm)

    pa, pb = _pair_tables()
    pa_arr = jnp.asarray(np.array(pa, np.int32))
    pb_arr = jnp.asarray(np.array(pb, np.int32))
    wf = weights.astype(jnp.float32)
    w1 = jnp.where(pa_arr == N, jnp.ones((NPROD,), jnp.float32),
                   wf[jnp.clip(pa_arr, 0, N - 1)])
    wprod = w1 * wf[pb_arr]
    wrow = jnp.repeat(wprod, D)[:, None]

    outT = _products(emb, wrow, pa_arr, pb_arr)
    return outT.T


# speedup vs baseline: 2.6647x; 2.6647x over previous
import jax
import jax.numpy as jnp
import numpy as np
from jax import lax
from jax.experimental import pallas as pl
from jax.experimental.pallas import tpu as pltpu
from jax.experimental.pallas import tpu_sc as plsc

NC = 2
NS = 16
L = 16

B = 4096
N = 20
D = 32
NPROD = N + N * (N - 1) // 2
OUT_F = NPROD * D
NW = NC * NS
BPW = B // NW
NSLOT = N + 1
EROWS = NSLOT * D
TCOL = 512
TBLK = 16384
PCHUNK = 30
NPC = NPROD // PCHUNK


def _pair_tables():
    pa, pb = [], []
    for i in range(N):
        pa.append(N)
        pb.append(i)
    for i in range(N):
        for j in range(i + 1, N):
            pa.append(i)
            pb.append(j)
    return pa, pb


def _t_body(src_ref, eye_ref, dst_ref):
    dst_ref[...] = lax.dot_general(
        src_ref[...], eye_ref[...], (((0,), (0,)), ((), ())),
        preferred_element_type=jnp.float32)


def _transpose_table(tabT, eye):
    m = tabT.shape[1]
    grid = (m + TBLK - 1) // TBLK
    return pl.pallas_call(
        _t_body,
        grid=(grid,),
        in_specs=[pl.BlockSpec((D, TBLK), lambda c: (0, c)),
                  pl.BlockSpec((D, D), lambda c: (0, 0))],
        out_specs=pl.BlockSpec((TBLK, D), lambda c: (c, 0)),
        out_shape=jax.ShapeDtypeStruct((m, D), jnp.float32),
    )(tabT, eye)


def _g_body(uidx_hbm, midxT_hbm, utab_hbm, itab_hbm, emb_hbm,
            midx_v, uidx_v, rows_v, ev_v, sem_g):
    w = lax.axis_index("s") * NC + lax.axis_index("c")
    lane0 = w * BPW

    pltpu.sync_copy(midxT_hbm.at[:, pl.ds(lane0, BPW)], midx_v)
    pltpu.sync_copy(uidx_hbm.at[pl.ds(lane0, BPW)], uidx_v)

    lane_iota = jnp.arange(L, dtype=jnp.int32)
    gsize = NSLOT // 3

    for g in range(3):
        for sl in range(gsize):
            s = g * gsize + sl
            if s < N:
                pltpu.async_copy(itab_hbm.at[midx_v.at[s]],
                                 rows_v.at[pl.ds(sl * BPW, BPW)], sem_g)
            else:
                pltpu.async_copy(utab_hbm.at[uidx_v],
                                 rows_v.at[pl.ds(sl * BPW, BPW)], sem_g)
        for _ in range(gsize):
            pltpu.make_async_copy(
                itab_hbm.at[midx_v.at[0]],
                rows_v.at[pl.ds(0, BPW)], sem_g).wait()

        def dbody(d, _, g=g):
            for sl in range(gsize):
                for blk in range(BPW // L):
                    rows = lane_iota + (sl * BPW + blk * L)
                    vals = plsc.load_gather(
                        rows_v, [rows, jnp.full((L,), 0, jnp.int32) + d])
                    ev_v[(g * gsize + sl) * D + d, pl.ds(blk * L, L)] = vals
            return _

        lax.fori_loop(0, D, dbody, None)

    pltpu.sync_copy(ev_v, emb_hbm.at[:, pl.ds(lane0, BPW)])


def _gather_emb(uidx, midxT, utab_rm, itab_rm):
    mesh = plsc.VectorSubcoreMesh(core_axis_name="c", subcore_axis_name="s")
    k = pl.kernel(
        _g_body,
        out_type=jax.ShapeDtypeStruct((EROWS, B), jnp.float32),
        mesh=mesh,
        compiler_params=pltpu.CompilerParams(use_tc_tiling_on_sc=False,
                                             needs_layout_passes=False),
        scratch_types=[
            pltpu.VMEM((N, BPW), jnp.int32),
            pltpu.VMEM((BPW,), jnp.int32),
            pltpu.VMEM((NSLOT // 3 * BPW, D), jnp.float32),
            pltpu.VMEM((EROWS, BPW), jnp.float32),
            pltpu.SemaphoreType.DMA,
        ],
    )
    return k(uidx, midxT, utab_rm, itab_rm)


def _p_body(pa_ref, pb_ref, emb_ref, wrow_ref, out_ref):
    pc = pl.program_id(1)
    for k in range(PCHUNK):
        p = pc * PCHUNK + k
        a0 = pa_ref[p] * D
        b0 = pb_ref[p] * D
        av = emb_ref[pl.ds(a0, D), :]
        bv = emb_ref[pl.ds(b0, D), :]
        out_ref[pl.ds(k * D, D), :] = av * bv * wrow_ref[pl.ds(k * D, D), :]


def _products(emb, wrow, pa_arr, pb_arr):
    grid_spec = pltpu.PrefetchScalarGridSpec(
        num_scalar_prefetch=2,
        grid=(B // TCOL, NPC),
        in_specs=[
            pl.BlockSpec((EROWS, TCOL), lambda bb, pc, *_: (0, bb)),
            pl.BlockSpec((PCHUNK * D, 1), lambda bb, pc, *_: (pc, 0)),
        ],
        out_specs=pl.BlockSpec((PCHUNK * D, TCOL),
                               lambda bb, pc, *_: (pc, bb)),
    )
    return pl.pallas_call(
        _p_body,
        grid_spec=grid_spec,
        out_shape=jax.ShapeDtypeStruct((OUT_F, B), jnp.float32),
    )(pa_arr, pb_arr, emb, wrow)


def kernel(user, memory, user_table, item_table, weights):
    uidx = user.reshape(-1).astype(jnp.int32)
    midxT = memory.T.astype(jnp.int32)

    eye = jnp.eye(D, dtype=jnp.float32)
    utab_rm = _transpose_table(user_table.astype(jnp.float32).T, eye)
    itab_rm = _transpose_table(item_table.astype(jnp.float32).T, eye)

    emb = _gather_emb(uidx, midxT, utab_rm, itab_rm)

    pa, pb = _pair_tables()
    pa_arr = jnp.asarray(np.array(pa, np.int32))
    pb_arr = jnp.asarray(np.array(pb, np.int32))
    wf = weights.astype(jnp.float32)
    w1 = jnp.where(pa_arr == N, jnp.ones((NPROD,), jnp.float32),
                   wf[jnp.clip(pa_arr, 0, N - 1)])
    wprod = w1 * wf[pb_arr]
    wrow = jnp.repeat(wprod, D)[:, None]

    outT = _products(emb, wrow, pa_arr, pb_arr)
    return outT.T


# speedup vs baseline: 3.2043x; 1.2025x over previous
import jax
import jax.numpy as jnp
import numpy as np
from jax import lax
from jax.experimental import pallas as pl
from jax.experimental.pallas import tpu as pltpu
from jax.experimental.pallas import tpu_sc as plsc

NC = 2
NS = 16
L = 16

B = 4096
N = 20
D = 32
NPROD = N + N * (N - 1) // 2
OUT_F = NPROD * D
NW = NC * NS
BPW = B // NW
NSLOT = N + 1
EROWS = NSLOT * D
TCOL = 512
TBLK = 16384
PCHUNK = 30
NPC = NPROD // PCHUNK


def _pair_tables():
    pa, pb = [], []
    for i in range(N):
        pa.append(N)
        pb.append(i)
    for i in range(N):
        for j in range(i + 1, N):
            pa.append(i)
            pb.append(j)
    return pa, pb


def _t_body(src_ref, eye_ref, dst_ref):
    dst_ref[...] = lax.dot_general(
        src_ref[...], eye_ref[...], (((0,), (0,)), ((), ())),
        preferred_element_type=jnp.float32)


def _transpose_table(tabT, eye):
    m = tabT.shape[1]
    grid = (m + TBLK - 1) // TBLK
    return pl.pallas_call(
        _t_body,
        grid=(grid,),
        in_specs=[pl.BlockSpec((D, TBLK), lambda c: (0, c)),
                  pl.BlockSpec((D, D), lambda c: (0, 0))],
        out_specs=pl.BlockSpec((TBLK, D), lambda c: (c, 0)),
        out_shape=jax.ShapeDtypeStruct((m, D), jnp.float32),
    )(tabT, eye)


def _g_body(uidx_hbm, midxT_hbm, utab_hbm, itab_hbm, emb_hbm,
            midx_v, uidx_v, rows_v, ev_v, sem_g):
    w = lax.axis_index("s") * NC + lax.axis_index("c")
    lane0 = w * BPW

    pltpu.sync_copy(midxT_hbm.at[:, pl.ds(lane0, BPW)], midx_v)
    pltpu.sync_copy(uidx_hbm.at[pl.ds(lane0, BPW)], uidx_v)

    lane_iota = jnp.arange(L, dtype=jnp.int32)
    gsize = NSLOT // 3

    for g in range(3):
        for sl in range(gsize):
            s = g * gsize + sl
            if s < N:
                pltpu.async_copy(itab_hbm.at[midx_v.at[s]],
                                 rows_v.at[pl.ds(sl * BPW, BPW)], sem_g)
            else:
                pltpu.async_copy(utab_hbm.at[uidx_v],
                                 rows_v.at[pl.ds(sl * BPW, BPW)], sem_g)
        for _ in range(gsize):
            pltpu.make_async_copy(
                itab_hbm.at[midx_v.at[0]],
                rows_v.at[pl.ds(0, BPW)], sem_g).wait()

        def dbody(d, _, g=g):
            for sl in range(gsize):
                for blk in range(BPW // L):
                    rows = lane_iota + (sl * BPW + blk * L)
                    vals = plsc.load_gather(
                        rows_v, [rows, jnp.full((L,), 0, jnp.int32) + d])
                    ev_v[(g * gsize + sl) * D + d, pl.ds(blk * L, L)] = vals
            return _

        lax.fori_loop(0, D, dbody, None)

    pltpu.sync_copy(ev_v, emb_hbm.at[:, pl.ds(lane0, BPW)])


def _gather_emb(uidx, midxT, utab_rm, itab_rm):
    mesh = plsc.VectorSubcoreMesh(core_axis_name="c", subcore_axis_name="s")
    k = pl.kernel(
        _g_body,
        out_type=jax.ShapeDtypeStruct((EROWS, B), jnp.float32),
        mesh=mesh,
        compiler_params=pltpu.CompilerParams(use_tc_tiling_on_sc=False,
                                             needs_layout_passes=False),
        scratch_types=[
            pltpu.VMEM((N, BPW), jnp.int32),
            pltpu.VMEM((BPW,), jnp.int32),
            pltpu.VMEM((NSLOT // 3 * BPW, D), jnp.float32),
            pltpu.VMEM((EROWS, BPW), jnp.float32),
            pltpu.SemaphoreType.DMA,
        ],
    )
    return k(uidx, midxT, utab_rm, itab_rm)


def _p_body(pa_ref, pb_ref, emb_ref, wrow_ref, out_ref):
    pc = pl.program_id(1)
    for k in range(PCHUNK):
        p = pc * PCHUNK + k
        a0 = pa_ref[p] * D
        b0 = pb_ref[p] * D
        av = emb_ref[pl.ds(a0, D), :]
        bv = emb_ref[pl.ds(b0, D), :]
        out_ref[pl.ds(k * D, D), :] = av * bv * wrow_ref[pl.ds(k * D, D), :]


def _products(emb, wrow, pa_arr, pb_arr):
    grid_spec = pltpu.PrefetchScalarGridSpec(
        num_scalar_prefetch=2,
        grid=(B // TCOL, NPC),
        in_specs=[
            pl.BlockSpec((EROWS, TCOL), lambda bb, pc, *_: (0, bb)),
            pl.BlockSpec((PCHUNK * D, 1), lambda bb, pc, *_: (pc, 0)),
        ],
        out_specs=pl.BlockSpec((PCHUNK * D, TCOL),
                               lambda bb, pc, *_: (pc, bb)),
    )
    return pl.pallas_call(
        _p_body,
        grid_spec=grid_spec,
        out_shape=jax.ShapeDtypeStruct((OUT_F, B), jnp.float32),
    )(pa_arr, pb_arr, emb, wrow)


def kernel(user, memory, user_table, item_table, weights):
    uidx = user.reshape(-1).astype(jnp.int32)
    midxT = memory.T.astype(jnp.int32)

    utab_rm = user_table.astype(jnp.float32)
    itab_rm = item_table.astype(jnp.float32)

    emb = _gather_emb(uidx, midxT, utab_rm, itab_rm)

    pa, pb = _pair_tables()
    pa_arr = jnp.asarray(np.array(pa, np.int32))
    pb_arr = jnp.asarray(np.array(pb, np.int32))
    wf = weights.astype(jnp.float32)
    w1 = jnp.where(pa_arr == N, jnp.ones((NPROD,), jnp.float32),
                   wf[jnp.clip(pa_arr, 0, N - 1)])
    wprod = w1 * wf[pb_arr]
    wrow = jnp.repeat(wprod, D)[:, None]

    outT = _products(emb, wrow, pa_arr, pb_arr)
    return outT.T
